# baseline (speedup 1.0000x reference)
import jax
import jax.numpy as jnp
from jax import lax
from jax.experimental import pallas as pl
from jax.experimental.pallas import tpu as pltpu
from jax.experimental.pallas import tpu_sc as plsc

VOCAB = 49408
NUM_POS = 77
POS_PAD = 80
EMBED_DIM = 768
BATCH = 1024
HALF = BATCH // 2

_INFO = plsc.get_sparse_core_info()
_NC = _INFO.num_cores
_NS = _INFO.num_subcores
_NW = _NC * _NS
_BPW = HALF // _NW


def _gather_body(tok_hbm, table_hbm, out_hbm, idx_v, spmem, sem_i, sem_o, sem_d):
    c = lax.axis_index("c")
    s = lax.axis_index("s")
    wid = s * _NC + c
    b0 = wid * _BPW

    pltpu.async_copy(
        tok_hbm.at[pl.ds(wid * _BPW * POS_PAD, _BPW * POS_PAD)], idx_v, sem_i
    ).wait()

    slot0 = s * (2 * POS_PAD)

    def batch(j, carry):
        base = slot0 + lax.rem(j, 2) * POS_PAD

        @pl.when(j >= 2)
        def _():
            pltpu.make_async_copy(
                spmem.at[pl.ds(slot0, POS_PAD), :],
                out_hbm.at[b0],
                sem_o,
            ).wait()

        for k in range(POS_PAD // 16):
            vec = idx_v[pl.ds(j * POS_PAD + k * 16, 16)]
            for l in range(16):
                p = k * 16 + l
                pltpu.async_copy(
                    table_hbm.at[pl.ds(vec[l], 1), :],
                    spmem.at[pl.ds(base + p, 1), :],
                    sem_d,
                )

        def drain_row(i, c2):
            pltpu.make_async_copy(
                table_hbm.at[pl.ds(0, 1), :],
                spmem.at[pl.ds(slot0, 1), :],
                sem_d,
            ).wait()
            return c2

        lax.fori_loop(0, POS_PAD, drain_row, 0)

        pltpu.async_copy(
            spmem.at[pl.ds(base, POS_PAD), :], out_hbm.at[b0 + j], sem_o
        )
        return carry

    lax.fori_loop(0, _BPW, batch, 0)

    def drain_out(i, c2):
        pltpu.make_async_copy(
            spmem.at[pl.ds(slot0, POS_PAD), :], out_hbm.at[b0], sem_o
        ).wait()
        return c2

    lax.fori_loop(0, 2, drain_out, 0)


def _add_body(g_ref, pos_ref, out_ref):
    out_ref[...] = g_ref[:, :NUM_POS, :] + pos_ref[...]


def _add_body2(_, g_ref, pos_ref, out_ref):
    out_ref[...] = g_ref[:, :NUM_POS, :] + pos_ref[...]


def _sc_gather(tok_half, token_table):
    mesh = plsc.VectorSubcoreMesh(core_axis_name="c", subcore_axis_name="s")
    return pl.kernel(
        _gather_body,
        mesh=mesh,
        out_type=jax.ShapeDtypeStruct((HALF, POS_PAD, EMBED_DIM), jnp.float32),
        scratch_types=[
            pltpu.VMEM((_BPW * POS_PAD,), jnp.int32),
            pltpu.VMEM_SHARED((_NS * 2 * POS_PAD, EMBED_DIM), jnp.float32),
            pltpu.SemaphoreType.DMA,
            pltpu.SemaphoreType.DMA,
            pltpu.SemaphoreType.DMA,
        ],
    )(tok_half, token_table)


@jax.jit
def kernel(input_tokens, token_table, pos_table):
    tok = jnp.pad(
        input_tokens.astype(jnp.int32), ((0, 0), (0, POS_PAD - NUM_POS))
    ).reshape(BATCH, POS_PAD)

    g1 = _sc_gather(tok[:HALF].reshape(HALF * POS_PAD), token_table)
    g2 = _sc_gather(tok[HALF:].reshape(HALF * POS_PAD), token_table)

    grid = (HALF // 8,)
    pos = pos_table[None]
    o1 = pl.pallas_call(
        _add_body,
        grid=grid,
        in_specs=[
            pl.BlockSpec((8, POS_PAD, EMBED_DIM), lambda i: (i, 0, 0)),
            pl.BlockSpec((1, NUM_POS, EMBED_DIM), lambda i: (0, 0, 0)),
        ],
        out_specs=pl.BlockSpec((8, NUM_POS, EMBED_DIM), lambda i: (i, 0, 0)),
        out_shape=jax.ShapeDtypeStruct((BATCH, NUM_POS, EMBED_DIM), jnp.float32),
    )(g1, pos)

    return pl.pallas_call(
        _add_body2,
        grid=grid,
        in_specs=[
            pl.BlockSpec((1, NUM_POS, 128), lambda i: (0, 0, 0)),
            pl.BlockSpec((8, POS_PAD, EMBED_DIM), lambda i: (i, 0, 0)),
            pl.BlockSpec((1, NUM_POS, EMBED_DIM), lambda i: (0, 0, 0)),
        ],
        out_specs=pl.BlockSpec(
            (8, NUM_POS, EMBED_DIM), lambda i: (i + HALF // 8, 0, 0)
        ),
        out_shape=jax.ShapeDtypeStruct((BATCH, NUM_POS, EMBED_DIM), jnp.float32),
        input_output_aliases={0: 0},
    )(o1, g2, pos)
